# Initial kernel scaffold; baseline (speedup 1.0000x reference)
import jax
import jax.numpy as jnp
from jax.experimental import pallas as pl


def kernel(x, edge_index, W1, b1, W2, b2, g1_Wl, g1_bl, g1_Wr, g1_br, g1_att, g1_bias, g2_Wl, g2_bl, g2_Wr, g2_br, g2_att, g2_bias):
    raise NotImplementedError("write your pallas kernel here")


# speedup vs baseline: 1.0033x; 1.0033x over previous
import functools

import jax
import jax.numpy as jnp
from jax.experimental import pallas as pl

N = 10000
E = 160000
FEAT = 5
HID = 128
OUT = 128
H1 = 4

ROW_BLK = 1000


def _proj1_body(x_ref, W1_ref, b1_ref, W2_ref, b2_ref, Wl_ref, bl_ref,
                Wr_ref, br_ref, xl_ref, xr_ref):
    x = x_ref[...]
    h = jnp.maximum(x @ W1_ref[...] + b1_ref[...], 0.0) @ W2_ref[...] + b2_ref[...]
    xl_ref[...] = h @ Wl_ref[...] + bl_ref[...]
    xr_ref[...] = h @ Wr_ref[...] + br_ref[...]


def _proj1(x, W1, b1, W2, b2, Wl, bl, Wr, br):
    nblk = N // ROW_BLK
    width = Wl.shape[1]
    return pl.pallas_call(
        _proj1_body,
        grid=(nblk,),
        in_specs=[
            pl.BlockSpec((ROW_BLK, FEAT), lambda i: (i, 0)),
            pl.BlockSpec((FEAT, HID), lambda i: (0, 0)),
            pl.BlockSpec((HID,), lambda i: (0,)),
            pl.BlockSpec((HID, HID), lambda i: (0, 0)),
            pl.BlockSpec((HID,), lambda i: (0,)),
            pl.BlockSpec((HID, width), lambda i: (0, 0)),
            pl.BlockSpec((width,), lambda i: (0,)),
            pl.BlockSpec((HID, width), lambda i: (0, 0)),
            pl.BlockSpec((width,), lambda i: (0,)),
        ],
        out_specs=[
            pl.BlockSpec((ROW_BLK, width), lambda i: (i, 0)),
            pl.BlockSpec((ROW_BLK, width), lambda i: (i, 0)),
        ],
        out_shape=[
            jax.ShapeDtypeStruct((N, width), jnp.float32),
            jax.ShapeDtypeStruct((N, width), jnp.float32),
        ],
    )(x, W1, b1, W2, b2, Wl, bl, Wr, br)


def _proj2_body(h_ref, bias_ref, Wl_ref, bl_ref, Wr_ref, br_ref, xl_ref, xr_ref):
    h = jnp.maximum(h_ref[...] + bias_ref[...], 0.0)
    xl_ref[...] = h @ Wl_ref[...] + bl_ref[...]
    xr_ref[...] = h @ Wr_ref[...] + br_ref[...]


def _proj2(h, bias, Wl, bl, Wr, br):
    nblk = N // ROW_BLK
    kin = Wl.shape[0]
    width = Wl.shape[1]
    return pl.pallas_call(
        _proj2_body,
        grid=(nblk,),
        in_specs=[
            pl.BlockSpec((ROW_BLK, kin), lambda i: (i, 0)),
            pl.BlockSpec((kin,), lambda i: (0,)),
            pl.BlockSpec((kin, width), lambda i: (0, 0)),
            pl.BlockSpec((width,), lambda i: (0,)),
            pl.BlockSpec((kin, width), lambda i: (0, 0)),
            pl.BlockSpec((width,), lambda i: (0,)),
        ],
        out_specs=[
            pl.BlockSpec((ROW_BLK, width), lambda i: (i, 0)),
            pl.BlockSpec((ROW_BLK, width), lambda i: (i, 0)),
        ],
        out_shape=[
            jax.ShapeDtypeStruct((N, width), jnp.float32),
            jax.ShapeDtypeStruct((N, width), jnp.float32),
        ],
    )(h, bias, Wl, bl, Wr, br)


def _edge_phase(xl, xr, src, dst, att, heads, out_dim):
    xlh = xl.reshape(N, heads, out_dim)
    xrh = xr.reshape(N, heads, out_dim)
    e = jax.nn.leaky_relu(xlh[src] + xrh[dst], negative_slope=0.2)
    alpha = jnp.sum(e * att[None, :, :], axis=-1)
    amax = jax.ops.segment_max(alpha, dst, num_segments=N)
    amax = jnp.where(jnp.isfinite(amax), amax, 0.0)
    ex = jnp.exp(alpha - amax[dst])
    denom = jax.ops.segment_sum(ex, dst, num_segments=N)
    a = ex / (denom[dst] + 1e-16)
    msg = xlh[src] * a[:, :, None]
    out = jax.ops.segment_sum(msg, dst, num_segments=N)
    return out.reshape(N, heads * out_dim)


def kernel(x, edge_index, W1, b1, W2, b2, g1_Wl, g1_bl, g1_Wr, g1_br,
           g1_att, g1_bias, g2_Wl, g2_bl, g2_Wr, g2_br, g2_att, g2_bias):
    loop = jnp.arange(N, dtype=edge_index.dtype)
    src = jnp.concatenate([edge_index[0], loop])
    dst = jnp.concatenate([edge_index[1], loop])

    xl1, xr1 = _proj1(x, W1, b1, W2, b2, g1_Wl, g1_bl, g1_Wr, g1_br)
    g1 = _edge_phase(xl1, xr1, src, dst, g1_att, H1, HID)
    xl2, xr2 = _proj2(g1, g1_bias, g2_Wl, g2_bl, g2_Wr, g2_br)
    g2 = _edge_phase(xl2, xr2, src, dst, g2_att, 1, OUT)
    return g2 + g2_bias
